# baseline (device time: 12540 ns/iter reference)
import jax
import jax.numpy as jnp
from jax import lax
from jax.experimental import pallas as pl
from jax.experimental.pallas import tpu as pltpu

X, Y, Z = 2, 2, 4
N_DEV = X * Y * Z
N_REP = X * Z
NC = 4


def kernel(x, dy, gamma):
    m, d = x.shape
    rows = m // N_REP
    ck = rows // NC

    def body(x_hbm, dy_hbm, out_ref, xv, dyv, acc_ref, comm_ref,
             in_sems, loc_sem, send_sems, recv_sems):
        my_x = lax.axis_index("x")
        my_y = lax.axis_index("y")
        my_z = lax.axis_index("z")
        r = my_x * Z + my_z
        my_lin = (my_x * Y + my_y) * Z + my_z

        start = r * rows
        cps = []
        for c in range(NC):
            cp_x = pltpu.make_async_copy(
                x_hbm.at[pl.ds(start + c * ck, ck), :],
                xv.at[pl.ds(c * ck, ck), :], in_sems.at[2 * c])
            cp_dy = pltpu.make_async_copy(
                dy_hbm.at[pl.ds(start + c * ck, ck), :],
                dyv.at[pl.ds(c * ck, ck), :], in_sems.at[2 * c + 1])
            cp_x.start()
            cp_dy.start()
            cps.append((cp_x, cp_dy))

        barrier = pltpu.get_barrier_semaphore()
        for px in range(X):
            for py in range(Y):
                for pz in range(Z):
                    p_lin = (px * Y + py) * Z + pz

                    @pl.when(p_lin != my_lin)
                    def _():
                        pl.semaphore_signal(
                            barrier, inc=1,
                            device_id=(px, py, pz),
                            device_id_type=pl.DeviceIdType.MESH,
                        )

        dgamma = jnp.zeros((1, d), jnp.float32)
        dbeta = jnp.zeros((1, d), jnp.float32)
        for c, (cp_x, cp_dy) in enumerate(cps):
            cp_x.wait()
            cp_dy.wait()
            xb = xv[pl.ds(c * ck, ck), :]
            dyb = dyv[pl.ds(c * ck, ck), :]
            mu = jnp.mean(xb, axis=1, keepdims=True)
            var = jnp.mean(xb * xb, axis=1, keepdims=True) - mu * mu
            rstd = lax.rsqrt(var + 1e-5)
            xhat = (xb - mu) * rstd
            dgamma = dgamma + jnp.sum(dyb * xhat, axis=0, keepdims=True)
            dbeta = dbeta + jnp.sum(dyb, axis=0, keepdims=True)
        acc_ref[...] = jnp.concatenate([dgamma, dbeta], axis=0)

        pl.semaphore_wait(barrier, N_DEV - 1)

        loc = pltpu.make_async_copy(acc_ref, comm_ref.at[my_lin], loc_sem)
        loc.start()
        rdmas = []
        for px in range(X):
            for py in range(Y):
                for pz in range(Z):
                    p_lin = (px * Y + py) * Z + pz
                    rdma = pltpu.make_async_remote_copy(
                        src_ref=acc_ref,
                        dst_ref=comm_ref.at[my_lin],
                        send_sem=send_sems.at[p_lin],
                        recv_sem=recv_sems.at[my_lin],
                        device_id=(px, py, pz),
                        device_id_type=pl.DeviceIdType.MESH,
                    )
                    rdmas.append((p_lin, rdma))

                    @pl.when(p_lin != my_lin)
                    def _():
                        rdma.start()

        for px in range(X):
            for py in range(Y):
                for pz in range(Z):
                    p_lin = (px * Y + py) * Z + pz
                    recv = pltpu.make_async_remote_copy(
                        src_ref=acc_ref,
                        dst_ref=comm_ref.at[p_lin],
                        send_sem=send_sems.at[p_lin],
                        recv_sem=recv_sems.at[p_lin],
                        device_id=(px, py, pz),
                        device_id_type=pl.DeviceIdType.MESH,
                    )

                    @pl.when(p_lin != my_lin)
                    def _():
                        recv.wait_recv()

        loc.wait()
        out_ref[...] = jnp.sum(comm_ref[...], axis=0)

        for p_lin, rdma in rdmas:
            @pl.when(p_lin != my_lin)
            def _():
                rdma.wait_send()

    return pl.pallas_call(
        body,
        in_specs=[
            pl.BlockSpec(memory_space=pl.ANY),
            pl.BlockSpec(memory_space=pl.ANY),
        ],
        out_specs=pl.BlockSpec(memory_space=pltpu.VMEM),
        out_shape=jax.ShapeDtypeStruct((2, d), jnp.float32),
        scratch_shapes=[
            pltpu.VMEM((rows, d), jnp.float32),
            pltpu.VMEM((rows, d), jnp.float32),
            pltpu.VMEM((2, d), jnp.float32),
            pltpu.VMEM((N_DEV, 2, d), jnp.float32),
            pltpu.SemaphoreType.DMA((2 * NC,)),
            pltpu.SemaphoreType.DMA,
            pltpu.SemaphoreType.DMA((N_DEV,)),
            pltpu.SemaphoreType.DMA((N_DEV,)),
        ],
        compiler_params=pltpu.CompilerParams(
            collective_id=0,
        ),
    )(x, dy)
